# baseline (speedup 1.0000x reference)
import functools

import jax
import jax.numpy as jnp
from jax import lax
from jax.experimental import pallas as pl
from jax.experimental.pallas import tpu as pltpu
from jax.experimental.pallas import tpu_sc as plsc

_LARGE_NEG = -100000.0
_BSZ, _NP, _NL = 128, 200, 48
_NW = 32
_RPW = _BSZ // _NW
_NCH = 13
_SEG = 256
_NEG = -3.0e38
_LN2 = 0.6931471805599453

def _gumbel_const():
    return jax.random.gumbel(
        jax.random.key(42), (_BSZ, _NP), jnp.float32).reshape(-1)

_GDN = lax.GatherDimensionNumbers(
    offset_dims=(), collapsed_slice_dims=(0,), start_index_map=(0,))


def _perm(x, idx):
    return lax.gather(x, idx[:, None], _GDN, (1,),
                      mode=lax.GatherScatterMode.PROMISE_IN_BOUNDS)


def _bfly(x, op, lane):
    for d in (8, 4, 2, 1):
        x = op(x, _perm(x, lane ^ d))
    return x


@functools.partial(
    pl.kernel,
    out_type=[
        jax.ShapeDtypeStruct((_NW, 128), jnp.int32),
        jax.ShapeDtypeStruct((_NW, 128), jnp.float32),
    ],
    mesh=plsc.VectorSubcoreMesh(core_axis_name="c", subcore_axis_name="s"),
    scratch_types=[
        pltpu.VMEM((_RPW * _NP,), jnp.int32),
        pltpu.VMEM((_RPW * _NP,), jnp.float32),
        pltpu.VMEM((_RPW * _NP,), jnp.float32),
        pltpu.VMEM((_RPW * _SEG,), jnp.int32),
        pltpu.VMEM((_RPW * _SEG,), jnp.float32),
        pltpu.VMEM((128,), jnp.int32),
        pltpu.VMEM((128,), jnp.float32),
        pltpu.SemaphoreType.DMA,
        pltpu.SemaphoreType.DMA,
    ],
)
def _sched(flat_hbm, li_hbm, mk_hbm, gm_hbm, act_hbm, lp_hbm,
           li_v, mk_v, gm_v, idx_v, gat_v, outa_v, outl_v, sem, sem2):
    wid = lax.axis_index("s") * 2 + lax.axis_index("c")
    base = wid * (_RPW * _NP)
    lane = lax.iota(jnp.int32, 16)
    lane48 = lane * _NL
    shift8 = (lane + 8) & 15

    pltpu.sync_copy(li_hbm.at[pl.ds(base, _RPW * _NP)], li_v)
    bg_mk = pltpu.async_copy(mk_hbm.at[pl.ds(base, _RPW * _NP)], mk_v, sem2)
    bg_gm = pltpu.async_copy(gm_hbm.at[pl.ds(base, _RPW * _NP)], gm_v, sem2)

    for r in range(_RPW):
        rowbase = (wid * _RPW + r) * (_NP * _NL)
        for c in range(_NCH):
            p0 = 16 * c
            if c < _NCH - 1:
                liv = li_v[pl.ds(_NP * r + p0, 16)]
                fi = (rowbase + p0 * _NL) + lane48 + liv
            else:
                lix = _perm(li_v[pl.ds(_NP * r + _NP - 16, 16)], shift8)
                fi = jnp.where(
                    lane < 8,
                    (rowbase + p0 * _NL) + lane48 + lix,
                    (rowbase + (p0 - _NP) * _NL) + lane48)
            idx_v[pl.ds(_SEG * r + p0, 16)] = fi

    handles = [
        pltpu.async_copy(flat_hbm.at[idx_v.at[pl.ds(128 * k, 128)]],
                         gat_v.at[pl.ds(128 * k, 128)], sem)
        for k in range(_RPW * _SEG // 128)
    ]
    bg_mk.wait()
    bg_gm.wait()
    for h in handles:
        h.wait()

    acc_a = jnp.zeros((16,), jnp.int32)
    acc_l = jnp.zeros((16,), jnp.float32)
    for r in range(_RPW):
        pmax = jnp.full((16,), _NEG, jnp.float32)
        bz = jnp.full((16,), _NEG, jnp.float32)
        bc = jnp.zeros((16,), jnp.int32)
        prios = []
        for c in range(_NCH):
            p0 = 16 * c
            gv = gat_v[pl.ds(_SEG * r + p0, 16)]
            if c < _NCH - 1:
                mv = mk_v[pl.ds(_NP * r + p0, 16)]
                gb = gm_v[pl.ds(_NP * r + p0, 16)]
                prio = gv + (1.0 - mv) * _LARGE_NEG
            else:
                mv = _perm(mk_v[pl.ds(_NP * r + _NP - 16, 16)], shift8)
                gb = _perm(gm_v[pl.ds(_NP * r + _NP - 16, 16)], shift8)
                prio = gv + (1.0 - mv) * _LARGE_NEG
                prio = jnp.where(lane < 8, prio, _NEG)
            prios.append(prio)
            pmax = jnp.maximum(pmax, prio)
            z = prio + gb
            upd = z > bz
            bc = jnp.where(upd, c, bc)
            bz = jnp.where(upd, z, bz)
        zmax = _bfly(bz, jnp.maximum, lane)
        cand = jnp.where(bz == zmax, bc * 16 + lane, 1 << 30)
        p_star = _bfly(cand, jnp.minimum, lane)
        pmaxs = _bfly(pmax, jnp.maximum, lane)
        sv = jnp.zeros((16,), jnp.float32)
        pa = jnp.full((16,), _NEG, jnp.float32)
        for c in range(_NCH):
            sv = sv + jnp.exp(prios[c] - pmaxs)
            pa = jnp.where((lane + c * 16) == p_star, prios[c], pa)
        pa = _bfly(pa, jnp.maximum, lane)
        vs = _bfly(sv, jnp.add, lane)
        ebits = lax.shift_right_arithmetic(
            lax.bitcast_convert_type(vs, jnp.int32), 23) - 127
        y = ebits.astype(jnp.float32) * _LN2 + 0.375
        for _ in range(4):
            y = y + vs * jnp.exp(-y) - 1.0
        lp = pa - pmaxs - y
        is_r = lane == r
        acc_a = jnp.where(is_r, p_star, acc_a)
        acc_l = jnp.where(is_r, lp, acc_l)

    for t in range(8):
        outa_v[pl.ds(t * 16, 16)] = acc_a if t == 0 else jnp.zeros((16,), jnp.int32)
        outl_v[pl.ds(t * 16, 16)] = acc_l if t == 0 else jnp.zeros((16,), jnp.float32)
    pltpu.sync_copy(outa_v, act_hbm.at[wid])
    pltpu.sync_copy(outl_v, lp_hbm.at[wid])


def kernel(all_has_answer_logits, layer_indices, masks, init_priorities):
    del init_priorities
    bsz = all_has_answer_logits.shape[0]
    act2, lp2 = _sched(all_has_answer_logits.reshape(-1),
                       layer_indices.reshape(-1),
                       masks.reshape(-1),
                       _gumbel_const())
    return (act2[:, :_RPW].reshape(bsz), lp2[:, :_RPW].reshape(bsz))


# speedup vs baseline: 1.1966x; 1.1966x over previous
import functools

import jax
import jax.numpy as jnp
from jax import lax
from jax.experimental import pallas as pl
from jax.experimental.pallas import tpu as pltpu
from jax.experimental.pallas import tpu_sc as plsc

_LARGE_NEG = -100000.0
_BSZ, _NP, _NL = 128, 200, 48
_NW = 32
_RPW = _BSZ // _NW
_NCH = 13
_SEG = 256
_NEG = -3.0e38
_LN2 = 0.6931471805599453

def _gumbel_const():
    return jax.random.gumbel(
        jax.random.key(42), (_BSZ, _NP), jnp.float32).reshape(-1)

_GDN = lax.GatherDimensionNumbers(
    offset_dims=(), collapsed_slice_dims=(0,), start_index_map=(0,))


def _perm(x, idx):
    return lax.gather(x, idx[:, None], _GDN, (1,),
                      mode=lax.GatherScatterMode.PROMISE_IN_BOUNDS)


def _bfly(x, op, lane):
    for d in (8, 4, 2, 1):
        x = op(x, _perm(x, lane ^ d))
    return x


@functools.partial(
    pl.kernel,
    out_type=[
        jax.ShapeDtypeStruct((_NW, 128), jnp.int32),
        jax.ShapeDtypeStruct((_NW, 128), jnp.float32),
    ],
    mesh=plsc.VectorSubcoreMesh(core_axis_name="c", subcore_axis_name="s"),
    compiler_params=pltpu.CompilerParams(needs_layout_passes=False),
    scratch_types=[
        pltpu.VMEM((_RPW * _NP,), jnp.int32),
        pltpu.VMEM((_RPW * _NP,), jnp.float32),
        pltpu.VMEM((_RPW * _NP,), jnp.float32),
        pltpu.VMEM((_RPW, _NP, _NL), jnp.float32),
        pltpu.VMEM((128,), jnp.int32),
        pltpu.VMEM((128,), jnp.float32),
        pltpu.SemaphoreType.DMA,
        pltpu.SemaphoreType.DMA,
    ],
)
def _sched(logits_hbm, li_hbm, mk_hbm, gm_hbm, act_hbm, lp_hbm,
           li_v, mk_v, gm_v, log_v, outa_v, outl_v, sem, sem2):
    wid = lax.axis_index("s") * 2 + lax.axis_index("c")
    row0 = wid * _RPW
    base = wid * (_RPW * _NP)
    lane = lax.iota(jnp.int32, 16)
    shift8 = (lane + 8) & 15

    bg_lg = pltpu.async_copy(logits_hbm.at[pl.ds(row0, _RPW)], log_v, sem)
    bg_li = pltpu.async_copy(li_hbm.at[pl.ds(base, _RPW * _NP)], li_v, sem2)
    bg_mk = pltpu.async_copy(mk_hbm.at[pl.ds(base, _RPW * _NP)], mk_v, sem2)
    bg_gm = pltpu.async_copy(gm_hbm.at[pl.ds(base, _RPW * _NP)], gm_v, sem2)
    bg_lg.wait()
    bg_li.wait()
    bg_mk.wait()
    bg_gm.wait()

    acc_a = jnp.zeros((16,), jnp.int32)
    acc_l = jnp.zeros((16,), jnp.float32)
    for r in range(_RPW):
        pmax = jnp.full((16,), _NEG, jnp.float32)
        bz = jnp.full((16,), _NEG, jnp.float32)
        bc = jnp.zeros((16,), jnp.int32)
        prios = []
        rsplat = jnp.full((16,), r, jnp.int32)
        for c in range(_NCH):
            p0 = 16 * c
            if c < _NCH - 1:
                liv = li_v[pl.ds(_NP * r + p0, 16)]
                mv = mk_v[pl.ds(_NP * r + p0, 16)]
                gb = gm_v[pl.ds(_NP * r + p0, 16)]
                gv = plsc.load_gather(log_v, [rsplat, lane + p0, liv])
                prio = gv + (1.0 - mv) * _LARGE_NEG
            else:
                lix = _perm(li_v[pl.ds(_NP * r + _NP - 16, 16)], shift8)
                mv = _perm(mk_v[pl.ds(_NP * r + _NP - 16, 16)], shift8)
                gb = _perm(gm_v[pl.ds(_NP * r + _NP - 16, 16)], shift8)
                pv = jnp.where(lane < 8, lane + p0, _NP - 1)
                gv = plsc.load_gather(log_v, [rsplat, pv, lix])
                prio = gv + (1.0 - mv) * _LARGE_NEG
                prio = jnp.where(lane < 8, prio, _NEG)
            prios.append(prio)
            pmax = jnp.maximum(pmax, prio)
            z = prio + gb
            upd = z > bz
            bc = jnp.where(upd, c, bc)
            bz = jnp.where(upd, z, bz)
        zmax = _bfly(bz, jnp.maximum, lane)
        cand = jnp.where(bz == zmax, bc * 16 + lane, 1 << 30)
        p_star = _bfly(cand, jnp.minimum, lane)
        pmaxs = _bfly(pmax, jnp.maximum, lane)
        sv = jnp.zeros((16,), jnp.float32)
        pa = jnp.full((16,), _NEG, jnp.float32)
        for c in range(_NCH):
            sv = sv + jnp.exp(prios[c] - pmaxs)
            pa = jnp.where((lane + c * 16) == p_star, prios[c], pa)
        pa = _bfly(pa, jnp.maximum, lane)
        vs = _bfly(sv, jnp.add, lane)
        ebits = lax.shift_right_arithmetic(
            lax.bitcast_convert_type(vs, jnp.int32), 23) - 127
        y = ebits.astype(jnp.float32) * _LN2 + 0.375
        for _ in range(4):
            y = y + vs * jnp.exp(-y) - 1.0
        lp = pa - pmaxs - y
        is_r = lane == r
        acc_a = jnp.where(is_r, p_star, acc_a)
        acc_l = jnp.where(is_r, lp, acc_l)

    for t in range(8):
        outa_v[pl.ds(t * 16, 16)] = acc_a if t == 0 else jnp.zeros((16,), jnp.int32)
        outl_v[pl.ds(t * 16, 16)] = acc_l if t == 0 else jnp.zeros((16,), jnp.float32)
    pltpu.sync_copy(outa_v, act_hbm.at[wid])
    pltpu.sync_copy(outl_v, lp_hbm.at[wid])


def kernel(all_has_answer_logits, layer_indices, masks, init_priorities):
    del init_priorities
    bsz = all_has_answer_logits.shape[0]
    act2, lp2 = _sched(all_has_answer_logits,
                       layer_indices.reshape(-1),
                       masks.reshape(-1),
                       _gumbel_const())
    return (act2[:, :_RPW].reshape(bsz), lp2[:, :_RPW].reshape(bsz))


# speedup vs baseline: 1.7846x; 1.4914x over previous
import functools

import jax
import jax.numpy as jnp
from jax import lax
from jax.experimental import pallas as pl
from jax.experimental.pallas import tpu as pltpu
from jax.experimental.pallas import tpu_sc as plsc

_LARGE_NEG = -100000.0
_BSZ, _NP, _NL = 128, 200, 48
_MAXP = 16
_NG = 4
_NEG = -3.0e38
_LN2 = 0.6931471805599453


@functools.partial(
    pl.kernel,
    out_type=[
        jax.ShapeDtypeStruct((_BSZ,), jnp.int32),
        jax.ShapeDtypeStruct((_BSZ,), jnp.float32),
    ],
    mesh=plsc.VectorSubcoreMesh(core_axis_name="c", subcore_axis_name="s"),
    compiler_params=pltpu.CompilerParams(needs_layout_passes=False),
    scratch_types=[
        pltpu.VMEM((_MAXP, _NL, _BSZ), jnp.float32),
        pltpu.VMEM((_MAXP, _BSZ), jnp.int32),
        pltpu.VMEM((_MAXP, _BSZ), jnp.float32),
        pltpu.VMEM((_MAXP, _BSZ), jnp.float32),
        pltpu.VMEM((_MAXP * _NG * 16,), jnp.float32),
        pltpu.VMEM((_NG * 4 * 16,), jnp.float32),
        pltpu.VMEM((_NG * 16,), jnp.int32),
        pltpu.VMEM((16 * _NG * 4 * 16,), jnp.float32),
        pltpu.VMEM((16 * _NG * 16,), jnp.int32),
        pltpu.VMEM((16,), jnp.int32),
        pltpu.VMEM((16,), jnp.float32),
        pltpu.VMEM_SHARED((16 * _NG * 4 * 16,), jnp.float32),
        pltpu.VMEM_SHARED((16 * _NG * 16,), jnp.int32),
        pltpu.SemaphoreType.DMA,
        pltpu.SemaphoreType.DMA,
    ],
)
def _sched(lt_hbm, li_hbm, mk_hbm, gm_hbm, act_hbm, lp_hbm,
           log_v, li_v, mk_v, gm_v, prio_v, pf_v, pi_v, tf_v, ti_v,
           oa_v, ol_v, shf, shi, sem, sem2):
    core = lax.axis_index("c")
    sub = lax.axis_index("s")
    start = 8 * ((25 * sub) // 16)
    bound = 8 * ((25 * (sub + 1)) // 16)
    lane = lax.iota(jnp.int32, 16)

    bg_lg = pltpu.async_copy(
        lt_hbm.at[pl.ds(start, _MAXP), pl.ds(0, _NL), pl.ds(0, _BSZ)],
        log_v, sem)
    bg_li = pltpu.async_copy(
        li_hbm.at[pl.ds(start, _MAXP), pl.ds(0, _BSZ)], li_v, sem2)
    bg_mk = pltpu.async_copy(
        mk_hbm.at[pl.ds(start, _MAXP), pl.ds(0, _BSZ)], mk_v, sem2)
    bg_gm = pltpu.async_copy(
        gm_hbm.at[pl.ds(start, _MAXP), pl.ds(0, _BSZ)], gm_v, sem2)
    bg_li.wait()
    bg_mk.wait()
    bg_gm.wait()
    bg_lg.wait()

    bz = [jnp.full((16,), _NEG, jnp.float32) for _ in range(_NG)]
    bp = [jnp.zeros((16,), jnp.int32) for _ in range(_NG)]
    bpa = [jnp.full((16,), _NEG, jnp.float32) for _ in range(_NG)]
    pmax = [jnp.full((16,), _NEG, jnp.float32) for _ in range(_NG)]
    for g in range(_NG):
        b_idx = core * 64 + g * 16 + lane
        for i in range(_MAXP):
            isplat = jnp.full((16,), i, jnp.int32)
            liv = plsc.load_gather(li_v, [isplat, b_idx])
            mv = plsc.load_gather(mk_v, [isplat, b_idx])
            gb = plsc.load_gather(gm_v, [isplat, b_idx])
            gv = plsc.load_gather(log_v, [isplat, liv, b_idx])
            prio = gv + (1.0 - mv) * _LARGE_NEG
            prio = jnp.where(start + i < bound, prio, _NEG)
            prio_v[pl.ds((i * _NG + g) * 16, 16)] = prio
            z = prio + gb
            upd = z > bz[g]
            bz[g] = jnp.where(upd, z, bz[g])
            bp[g] = jnp.where(upd, start + i, bp[g])
            bpa[g] = jnp.where(upd, prio, bpa[g])
            pmax[g] = jnp.maximum(pmax[g], prio)

    for g in range(_NG):
        sv = jnp.zeros((16,), jnp.float32)
        for i in range(_MAXP):
            sv = sv + jnp.exp(prio_v[pl.ds((i * _NG + g) * 16, 16)] - pmax[g])
        pf_v[pl.ds((g * 4 + 0) * 16, 16)] = bz[g]
        pf_v[pl.ds((g * 4 + 1) * 16, 16)] = bpa[g]
        pf_v[pl.ds((g * 4 + 2) * 16, 16)] = pmax[g]
        pf_v[pl.ds((g * 4 + 3) * 16, 16)] = sv
        pi_v[pl.ds(g * 16, 16)] = bp[g]
    pltpu.sync_copy(pf_v, shf.at[pl.ds(sub * (_NG * 4 * 16), _NG * 4 * 16)])
    pltpu.sync_copy(pi_v, shi.at[pl.ds(sub * (_NG * 16), _NG * 16)])
    plsc.subcore_barrier()

    @pl.when(sub < _NG)
    def _combine():
        pltpu.sync_copy(shf, tf_v)
        pltpu.sync_copy(shi, ti_v)
        bz_g = jnp.full((16,), _NEG, jnp.float32)
        bp_g = jnp.zeros((16,), jnp.int32)
        bpa_g = jnp.full((16,), _NEG, jnp.float32)
        pmax_g = jnp.full((16,), _NEG, jnp.float32)
        parts = []
        for j in range(16):
            fbase = j * (_NG * 4 * 16) + sub * (4 * 16) + lane
            zj = plsc.load_gather(tf_v, [fbase])
            paj = plsc.load_gather(tf_v, [fbase + 16])
            pmj = plsc.load_gather(tf_v, [fbase + 32])
            svj = plsc.load_gather(tf_v, [fbase + 48])
            pj = plsc.load_gather(ti_v, [j * (_NG * 16) + sub * 16 + lane])
            upd = zj > bz_g
            bz_g = jnp.where(upd, zj, bz_g)
            bp_g = jnp.where(upd, pj, bp_g)
            bpa_g = jnp.where(upd, paj, bpa_g)
            pmax_g = jnp.maximum(pmax_g, pmj)
            parts.append((pmj, svj))
        s_g = jnp.zeros((16,), jnp.float32)
        for pmj, svj in parts:
            s_g = s_g + svj * jnp.exp(pmj - pmax_g)
        ebits = lax.shift_right_arithmetic(
            lax.bitcast_convert_type(s_g, jnp.int32), 23) - 127
        y = ebits.astype(jnp.float32) * _LN2 + 0.375
        for _ in range(4):
            y = y + s_g * jnp.exp(-y) - 1.0
        oa_v[...] = bp_g
        ol_v[...] = bpa_g - pmax_g - y
        out16 = (core * _NG + sub) * 16
        pltpu.sync_copy(oa_v, act_hbm.at[pl.ds(out16, 16)])
        pltpu.sync_copy(ol_v, lp_hbm.at[pl.ds(out16, 16)])


def kernel(all_has_answer_logits, layer_indices, masks, init_priorities):
    del init_priorities
    bsz, npass, _ = all_has_answer_logits.shape
    gum = jax.random.gumbel(jax.random.key(42), (bsz, npass), jnp.float32)
    act, lp = _sched(jnp.transpose(all_has_answer_logits, (1, 2, 0)),
                     layer_indices.T, masks.T, gum.T)
    return (act, lp)


# speedup vs baseline: 1.7904x; 1.0033x over previous
import functools

import jax
import jax.numpy as jnp
from jax import lax
from jax.experimental import pallas as pl
from jax.experimental.pallas import tpu as pltpu
from jax.experimental.pallas import tpu_sc as plsc

_LARGE_NEG = -100000.0
_BSZ, _NP, _NL = 128, 200, 48
_MAXP = 16
_NG = 4
_NEG = -3.0e38
_LN2 = 0.6931471805599453


@functools.partial(
    pl.kernel,
    out_type=[
        jax.ShapeDtypeStruct((_BSZ,), jnp.int32),
        jax.ShapeDtypeStruct((_BSZ,), jnp.float32),
    ],
    mesh=plsc.VectorSubcoreMesh(core_axis_name="c", subcore_axis_name="s"),
    compiler_params=pltpu.CompilerParams(needs_layout_passes=False),
    scratch_types=[
        pltpu.VMEM((_MAXP, _NL, _BSZ), jnp.float32),
        pltpu.VMEM((_MAXP, _BSZ), jnp.int32),
        pltpu.VMEM((_MAXP, _BSZ), jnp.float32),
        pltpu.VMEM((_MAXP, _BSZ), jnp.float32),
        pltpu.VMEM((_MAXP * _NG * 16,), jnp.float32),
        pltpu.VMEM((_NG * 4 * 16,), jnp.float32),
        pltpu.VMEM((_NG * 16,), jnp.int32),
        pltpu.VMEM((16 * _NG * 4 * 16,), jnp.float32),
        pltpu.VMEM((16 * _NG * 16,), jnp.int32),
        pltpu.VMEM((16,), jnp.int32),
        pltpu.VMEM((16,), jnp.float32),
        pltpu.VMEM_SHARED((16 * _NG * 4 * 16,), jnp.float32),
        pltpu.VMEM_SHARED((16 * _NG * 16,), jnp.int32),
        pltpu.SemaphoreType.DMA,
        pltpu.SemaphoreType.DMA,
    ],
)
def _sched(lt_hbm, li_hbm, mk_hbm, gm_hbm, act_hbm, lp_hbm,
           log_v, li_v, mk_v, gm_v, prio_v, pf_v, pi_v, tf_v, ti_v,
           oa_v, ol_v, shf, shi, sem, sem2):
    core = lax.axis_index("c")
    sub = lax.axis_index("s")
    start = 8 * ((25 * sub) // 16)
    bound = 8 * ((25 * (sub + 1)) // 16)
    lane = lax.iota(jnp.int32, 16)

    size = bound - start

    @pl.when(size == _MAXP)
    def _dma_full():
        pltpu.async_copy(
            lt_hbm.at[pl.ds(start, _MAXP), pl.ds(0, _NL), pl.ds(0, _BSZ)],
            log_v, sem)
        pltpu.async_copy(
            li_hbm.at[pl.ds(start, _MAXP), pl.ds(0, _BSZ)], li_v, sem2)
        pltpu.async_copy(
            mk_hbm.at[pl.ds(start, _MAXP), pl.ds(0, _BSZ)], mk_v, sem2)
        pltpu.async_copy(
            gm_hbm.at[pl.ds(start, _MAXP), pl.ds(0, _BSZ)], gm_v, sem2)

    @pl.when(size != _MAXP)
    def _dma_half():
        h = _MAXP // 2
        pltpu.async_copy(
            lt_hbm.at[pl.ds(start, h), pl.ds(0, _NL), pl.ds(0, _BSZ)],
            log_v.at[pl.ds(0, h)], sem)
        pltpu.async_copy(
            li_hbm.at[pl.ds(start, h), pl.ds(0, _BSZ)],
            li_v.at[pl.ds(0, h)], sem2)
        pltpu.async_copy(
            mk_hbm.at[pl.ds(start, h), pl.ds(0, _BSZ)],
            mk_v.at[pl.ds(0, h)], sem2)
        pltpu.async_copy(
            gm_hbm.at[pl.ds(start, h), pl.ds(0, _BSZ)],
            gm_v.at[pl.ds(0, h)], sem2)

    def _drain_half():
        pltpu.make_async_copy(
            lt_hbm.at[pl.ds(0, _MAXP // 2), pl.ds(0, _NL), pl.ds(0, _BSZ)],
            log_v.at[pl.ds(0, _MAXP // 2)], sem).wait()
        for _ in range(3):
            pltpu.make_async_copy(
                li_hbm.at[pl.ds(0, _MAXP // 2), pl.ds(0, _BSZ)],
                li_v.at[pl.ds(0, _MAXP // 2)], sem2).wait()

    _drain_half()

    @pl.when(size == _MAXP)
    def _drain_rest():
        _drain_half()

    bz = [jnp.full((16,), _NEG, jnp.float32) for _ in range(_NG)]
    bp = [jnp.zeros((16,), jnp.int32) for _ in range(_NG)]
    bpa = [jnp.full((16,), _NEG, jnp.float32) for _ in range(_NG)]
    pmax = [jnp.full((16,), _NEG, jnp.float32) for _ in range(_NG)]
    for g in range(_NG):
        b_idx = core * 64 + g * 16 + lane
        for i in range(_MAXP):
            isplat = jnp.full((16,), i, jnp.int32)
            liv = plsc.load_gather(li_v, [isplat, b_idx])
            if i >= _MAXP // 2:
                liv = jnp.where(start + i < bound, liv, 0)
            mv = plsc.load_gather(mk_v, [isplat, b_idx])
            gb = plsc.load_gather(gm_v, [isplat, b_idx])
            gv = plsc.load_gather(log_v, [isplat, liv, b_idx])
            prio = gv + (1.0 - mv) * _LARGE_NEG
            prio = jnp.where(start + i < bound, prio, _NEG)
            prio_v[pl.ds((i * _NG + g) * 16, 16)] = prio
            z = prio + gb
            upd = z > bz[g]
            bz[g] = jnp.where(upd, z, bz[g])
            bp[g] = jnp.where(upd, start + i, bp[g])
            bpa[g] = jnp.where(upd, prio, bpa[g])
            pmax[g] = jnp.maximum(pmax[g], prio)

    for g in range(_NG):
        sv = jnp.zeros((16,), jnp.float32)
        for i in range(_MAXP):
            sv = sv + jnp.exp(prio_v[pl.ds((i * _NG + g) * 16, 16)] - pmax[g])
        pf_v[pl.ds((g * 4 + 0) * 16, 16)] = bz[g]
        pf_v[pl.ds((g * 4 + 1) * 16, 16)] = bpa[g]
        pf_v[pl.ds((g * 4 + 2) * 16, 16)] = pmax[g]
        pf_v[pl.ds((g * 4 + 3) * 16, 16)] = sv
        pi_v[pl.ds(g * 16, 16)] = bp[g]
    pltpu.sync_copy(pf_v, shf.at[pl.ds(sub * (_NG * 4 * 16), _NG * 4 * 16)])
    pltpu.sync_copy(pi_v, shi.at[pl.ds(sub * (_NG * 16), _NG * 16)])
    plsc.subcore_barrier()

    @pl.when(sub < _NG)
    def _combine():
        pltpu.sync_copy(shf, tf_v)
        pltpu.sync_copy(shi, ti_v)
        bz_g = jnp.full((16,), _NEG, jnp.float32)
        bp_g = jnp.zeros((16,), jnp.int32)
        bpa_g = jnp.full((16,), _NEG, jnp.float32)
        pmax_g = jnp.full((16,), _NEG, jnp.float32)
        parts = []
        for j in range(16):
            fbase = j * (_NG * 4 * 16) + sub * (4 * 16) + lane
            zj = plsc.load_gather(tf_v, [fbase])
            paj = plsc.load_gather(tf_v, [fbase + 16])
            pmj = plsc.load_gather(tf_v, [fbase + 32])
            svj = plsc.load_gather(tf_v, [fbase + 48])
            pj = plsc.load_gather(ti_v, [j * (_NG * 16) + sub * 16 + lane])
            upd = zj > bz_g
            bz_g = jnp.where(upd, zj, bz_g)
            bp_g = jnp.where(upd, pj, bp_g)
            bpa_g = jnp.where(upd, paj, bpa_g)
            pmax_g = jnp.maximum(pmax_g, pmj)
            parts.append((pmj, svj))
        s_g = jnp.zeros((16,), jnp.float32)
        for pmj, svj in parts:
            s_g = s_g + svj * jnp.exp(pmj - pmax_g)
        ebits = lax.shift_right_arithmetic(
            lax.bitcast_convert_type(s_g, jnp.int32), 23) - 127
        y = ebits.astype(jnp.float32) * _LN2 + 0.375
        for _ in range(4):
            y = y + s_g * jnp.exp(-y) - 1.0
        oa_v[...] = bp_g
        ol_v[...] = bpa_g - pmax_g - y
        out16 = (core * _NG + sub) * 16
        pltpu.sync_copy(oa_v, act_hbm.at[pl.ds(out16, 16)])
        pltpu.sync_copy(ol_v, lp_hbm.at[pl.ds(out16, 16)])


def kernel(all_has_answer_logits, layer_indices, masks, init_priorities):
    del init_priorities
    bsz, npass, _ = all_has_answer_logits.shape
    gum = jax.random.gumbel(jax.random.key(42), (bsz, npass), jnp.float32)
    act, lp = _sched(jnp.transpose(all_has_answer_logits, (1, 2, 0)),
                     layer_indices.T, masks.T, gum.T)
    return (act, lp)


# speedup vs baseline: 1.7957x; 1.0029x over previous
import functools

import numpy as np

import jax
import jax.numpy as jnp
from jax import lax
from jax.experimental import pallas as pl
from jax.experimental.pallas import tpu as pltpu
from jax.experimental.pallas import tpu_sc as plsc

_LARGE_NEG = -100000.0
_BSZ, _NP, _NL = 128, 200, 48
_MAXP = 16
_NG = 4
_NEG = -3.0e38
_LN2 = 0.6931471805599453


@functools.partial(
    pl.kernel,
    out_type=[
        jax.ShapeDtypeStruct((_BSZ,), jnp.int32),
        jax.ShapeDtypeStruct((_BSZ,), jnp.float32),
    ],
    mesh=plsc.VectorSubcoreMesh(core_axis_name="c", subcore_axis_name="s"),
    compiler_params=pltpu.CompilerParams(needs_layout_passes=False),
    scratch_types=[
        pltpu.VMEM((_MAXP, _NL, _BSZ), jnp.float32),
        pltpu.VMEM((_MAXP, _BSZ), jnp.int32),
        pltpu.VMEM((_MAXP, _BSZ), jnp.float32),
        pltpu.VMEM((_MAXP, _BSZ), jnp.float32),
        pltpu.VMEM((_MAXP * _NG * 16,), jnp.float32),
        pltpu.VMEM((_NG * 4 * 16,), jnp.float32),
        pltpu.VMEM((_NG * 16,), jnp.int32),
        pltpu.VMEM((16 * _NG * 4 * 16,), jnp.float32),
        pltpu.VMEM((16 * _NG * 16,), jnp.int32),
        pltpu.VMEM((16,), jnp.int32),
        pltpu.VMEM((16,), jnp.float32),
        pltpu.VMEM_SHARED((16 * _NG * 4 * 16,), jnp.float32),
        pltpu.VMEM_SHARED((16 * _NG * 16,), jnp.int32),
        pltpu.SemaphoreType.DMA,
        pltpu.SemaphoreType.DMA,
    ],
)
def _sched(lt_hbm, li_hbm, mk_hbm, gm_hbm, act_hbm, lp_hbm,
           log_v, li_v, mk_v, gm_v, prio_v, pf_v, pi_v, tf_v, ti_v,
           oa_v, ol_v, shf, shi, sem, sem2):
    core = lax.axis_index("c")
    sub = lax.axis_index("s")
    start = 8 * ((25 * sub) // 16)
    bound = 8 * ((25 * (sub + 1)) // 16)
    lane = lax.iota(jnp.int32, 16)

    size = bound - start

    @pl.when(size == _MAXP)
    def _dma_full():
        pltpu.async_copy(
            lt_hbm.at[pl.ds(start, _MAXP), pl.ds(0, _NL), pl.ds(0, _BSZ)],
            log_v, sem)
        pltpu.async_copy(
            li_hbm.at[pl.ds(start, _MAXP), pl.ds(0, _BSZ)], li_v, sem2)
        pltpu.async_copy(
            mk_hbm.at[pl.ds(start, _MAXP), pl.ds(0, _BSZ)], mk_v, sem2)
        pltpu.async_copy(
            gm_hbm.at[pl.ds(start, _MAXP), pl.ds(0, _BSZ)], gm_v, sem2)

    @pl.when(size != _MAXP)
    def _dma_half():
        h = _MAXP // 2
        pltpu.async_copy(
            lt_hbm.at[pl.ds(start, h), pl.ds(0, _NL), pl.ds(0, _BSZ)],
            log_v.at[pl.ds(0, h)], sem)
        pltpu.async_copy(
            li_hbm.at[pl.ds(start, h), pl.ds(0, _BSZ)],
            li_v.at[pl.ds(0, h)], sem2)
        pltpu.async_copy(
            mk_hbm.at[pl.ds(start, h), pl.ds(0, _BSZ)],
            mk_v.at[pl.ds(0, h)], sem2)
        pltpu.async_copy(
            gm_hbm.at[pl.ds(start, h), pl.ds(0, _BSZ)],
            gm_v.at[pl.ds(0, h)], sem2)

    def _drain_half():
        pltpu.make_async_copy(
            lt_hbm.at[pl.ds(0, _MAXP // 2), pl.ds(0, _NL), pl.ds(0, _BSZ)],
            log_v.at[pl.ds(0, _MAXP // 2)], sem).wait()
        for _ in range(3):
            pltpu.make_async_copy(
                li_hbm.at[pl.ds(0, _MAXP // 2), pl.ds(0, _BSZ)],
                li_v.at[pl.ds(0, _MAXP // 2)], sem2).wait()

    _drain_half()

    @pl.when(size == _MAXP)
    def _drain_rest():
        _drain_half()

    bz = [jnp.full((16,), _NEG, jnp.float32) for _ in range(_NG)]
    bp = [jnp.zeros((16,), jnp.int32) for _ in range(_NG)]
    bpa = [jnp.full((16,), _NEG, jnp.float32) for _ in range(_NG)]
    pmax = [jnp.full((16,), _NEG, jnp.float32) for _ in range(_NG)]
    for g in range(_NG):
        b_idx = core * 64 + g * 16 + lane
        for i in range(_MAXP):
            isplat = jnp.full((16,), i, jnp.int32)
            liv = plsc.load_gather(li_v, [isplat, b_idx])
            if i >= _MAXP // 2:
                liv = jnp.where(start + i < bound, liv, 0)
            mv = plsc.load_gather(mk_v, [isplat, b_idx])
            gb = plsc.load_gather(gm_v, [isplat, b_idx])
            gv = plsc.load_gather(log_v, [isplat, liv, b_idx])
            prio = gv + (1.0 - mv) * _LARGE_NEG
            prio = jnp.where(start + i < bound, prio, _NEG)
            prio_v[pl.ds((i * _NG + g) * 16, 16)] = prio
            z = prio + gb
            upd = z > bz[g]
            bz[g] = jnp.where(upd, z, bz[g])
            bp[g] = jnp.where(upd, start + i, bp[g])
            bpa[g] = jnp.where(upd, prio, bpa[g])
            pmax[g] = jnp.maximum(pmax[g], prio)

    for g in range(_NG):
        sv = jnp.zeros((16,), jnp.float32)
        for i in range(_MAXP):
            sv = sv + jnp.exp(prio_v[pl.ds((i * _NG + g) * 16, 16)] - pmax[g])
        pf_v[pl.ds((g * 4 + 0) * 16, 16)] = bz[g]
        pf_v[pl.ds((g * 4 + 1) * 16, 16)] = bpa[g]
        pf_v[pl.ds((g * 4 + 2) * 16, 16)] = pmax[g]
        pf_v[pl.ds((g * 4 + 3) * 16, 16)] = sv
        pi_v[pl.ds(g * 16, 16)] = bp[g]
    pltpu.sync_copy(pf_v, shf.at[pl.ds(sub * (_NG * 4 * 16), _NG * 4 * 16)])
    pltpu.sync_copy(pi_v, shi.at[pl.ds(sub * (_NG * 16), _NG * 16)])
    plsc.subcore_barrier()

    @pl.when(sub < _NG)
    def _combine():
        pltpu.sync_copy(shf, tf_v)
        pltpu.sync_copy(shi, ti_v)
        bz_g = jnp.full((16,), _NEG, jnp.float32)
        bp_g = jnp.zeros((16,), jnp.int32)
        bpa_g = jnp.full((16,), _NEG, jnp.float32)
        pmax_g = jnp.full((16,), _NEG, jnp.float32)
        parts = []
        for j in range(16):
            fbase = j * (_NG * 4 * 16) + sub * (4 * 16) + lane
            zj = plsc.load_gather(tf_v, [fbase])
            paj = plsc.load_gather(tf_v, [fbase + 16])
            pmj = plsc.load_gather(tf_v, [fbase + 32])
            svj = plsc.load_gather(tf_v, [fbase + 48])
            pj = plsc.load_gather(ti_v, [j * (_NG * 16) + sub * 16 + lane])
            upd = zj > bz_g
            bz_g = jnp.where(upd, zj, bz_g)
            bp_g = jnp.where(upd, pj, bp_g)
            bpa_g = jnp.where(upd, paj, bpa_g)
            pmax_g = jnp.maximum(pmax_g, pmj)
            parts.append((pmj, svj))
        s_g = jnp.zeros((16,), jnp.float32)
        for pmj, svj in parts:
            s_g = s_g + svj * jnp.exp(pmj - pmax_g)
        ebits = lax.shift_right_arithmetic(
            lax.bitcast_convert_type(s_g, jnp.int32), 23) - 127
        y = ebits.astype(jnp.float32) * _LN2 + 0.375
        for _ in range(4):
            y = y + s_g * jnp.exp(-y) - 1.0
        oa_v[...] = bp_g
        ol_v[...] = bpa_g - pmax_g - y
        out16 = (core * _NG + sub) * 16
        pltpu.sync_copy(oa_v, act_hbm.at[pl.ds(out16, 16)])
        pltpu.sync_copy(ol_v, lp_hbm.at[pl.ds(out16, 16)])


_GUM = np.asarray(
    jax.random.gumbel(jax.random.key(42), (_BSZ, _NP), jnp.float32)).T.copy()


def kernel(all_has_answer_logits, layer_indices, masks, init_priorities):
    del init_priorities
    act, lp = _sched(jnp.transpose(all_has_answer_logits, (1, 2, 0)),
                     layer_indices.T, masks.T, jnp.asarray(_GUM))
    return (act, lp)
